# baseline (device time: 33343 ns/iter reference)
import jax
import jax.numpy as jnp
from jax import lax
from jax.experimental import pallas as pl
from jax.experimental.pallas import tpu as pltpu

N_CHUNKS = 4


def kernel(x, pi):
    _, m, n = x.shape
    rows = m // N_CHUNKS

    def body(
        x_ref, pi_ref, out_ref,
        xin, oout, send_q, recv_q, send_s, recv_s,
        in_sems, out_sems,
        qsend_sems, qrecv_sems, ssend_sems, srecv_sems,
    ):
        my_x = lax.axis_index("x")
        my_y = lax.axis_index("y")
        my_z = lax.axis_index("z")
        dst_y = pi_ref[my_y]
        src_y = jnp.int32(0)
        for k in range(4):
            src_y = jnp.where(pi_ref[k] == my_y, jnp.int32(k), src_y)

        def copy_in(c):
            sl = pl.ds(c * rows, rows)
            return pltpu.make_async_copy(
                x_ref.at[0, sl, :], xin.at[sl], in_sems.at[c]
            )

        def copy_out(c):
            sl = pl.ds(c * rows, rows)
            return pltpu.make_async_copy(
                oout.at[sl], out_ref.at[0, sl, :], out_sems.at[c]
            )

        for c in range(N_CHUNKS):
            copy_in(c).start()

        barrier = pltpu.get_barrier_semaphore()
        pl.semaphore_signal(
            barrier, inc=1, device_id=(my_x, dst_y, my_z),
            device_id_type=pl.DeviceIdType.MESH,
        )
        pl.semaphore_signal(
            barrier, inc=1, device_id=(my_x, src_y, my_z),
            device_id_type=pl.DeviceIdType.MESH,
        )

        def chunk_rdma(c):
            sl = pl.ds(c * rows, rows)
            return pltpu.make_async_remote_copy(
                src_ref=send_q.at[sl],
                dst_ref=recv_q.at[sl],
                send_sem=qsend_sems.at[c],
                recv_sem=qrecv_sems.at[c],
                device_id=(my_x, dst_y, my_z),
                device_id_type=pl.DeviceIdType.MESH,
            )

        def scale_rdma(c):
            sl = pl.ds(c * 8, 8)
            return pltpu.make_async_remote_copy(
                src_ref=send_s.at[sl],
                dst_ref=recv_s.at[sl],
                send_sem=ssend_sems.at[c],
                recv_sem=srecv_sems.at[c],
                device_id=(my_x, dst_y, my_z),
                device_id_type=pl.DeviceIdType.MESH,
            )

        for c in range(N_CHUNKS):
            sl = pl.ds(c * rows, rows)
            copy_in(c).wait()
            xc = xin[sl, :]
            absmax = jnp.max(jnp.abs(xc))
            inv = jnp.where(absmax > 0, 127.0 / absmax, 0.0)
            send_q[sl, :] = jnp.round(xc * inv).astype(jnp.int8)
            send_s[pl.ds(c * 8, 8), :] = jnp.full(
                (8, 128), absmax * (1.0 / 127.0), jnp.float32
            )
            if c == 0:
                pl.semaphore_wait(barrier, 2)
            scale_rdma(c).start()
            chunk_rdma(c).start()

        for c in range(N_CHUNKS):
            sl = pl.ds(c * rows, rows)
            s_rdma = scale_rdma(c)
            s_rdma.wait_send()
            s_rdma.wait_recv()
            q_rdma = chunk_rdma(c)
            q_rdma.wait_send()
            q_rdma.wait_recv()
            sc = recv_s[pl.ds(c * 8, 1), pl.ds(0, 1)]
            oout[sl, :] = recv_q[sl, :].astype(jnp.float32) * sc
            copy_out(c).start()

        for c in range(N_CHUNKS):
            copy_out(c).wait()

    return pl.pallas_call(
        body,
        out_shape=jax.ShapeDtypeStruct(x.shape, x.dtype),
        in_specs=[
            pl.BlockSpec(memory_space=pl.ANY),
            pl.BlockSpec(memory_space=pltpu.SMEM),
        ],
        out_specs=pl.BlockSpec(memory_space=pl.ANY),
        scratch_shapes=[
            pltpu.VMEM((m, n), jnp.float32),
            pltpu.VMEM((m, n), jnp.float32),
            pltpu.VMEM((m, n), jnp.int8),
            pltpu.VMEM((m, n), jnp.int8),
            pltpu.VMEM((N_CHUNKS * 8, 128), jnp.float32),
            pltpu.VMEM((N_CHUNKS * 8, 128), jnp.float32),
            pltpu.SemaphoreType.DMA((N_CHUNKS,)),
            pltpu.SemaphoreType.DMA((N_CHUNKS,)),
            pltpu.SemaphoreType.DMA((N_CHUNKS,)),
            pltpu.SemaphoreType.DMA((N_CHUNKS,)),
            pltpu.SemaphoreType.DMA((N_CHUNKS,)),
            pltpu.SemaphoreType.DMA((N_CHUNKS,)),
        ],
        compiler_params=pltpu.CompilerParams(collective_id=0),
    )(x, pi)


# device time: 32889 ns/iter; 1.0138x vs baseline; 1.0138x over previous
import jax
import jax.numpy as jnp
from jax import lax
from jax.experimental import pallas as pl
from jax.experimental.pallas import tpu as pltpu

N_CHUNKS = 4


def kernel(x, pi):
    _, m, n = x.shape
    rows = m // N_CHUNKS

    def body(
        x_ref, pi_ref, out_ref,
        send_q, recv_q, send_s, recv_s,
        qsend_sems, qrecv_sems, ssend_sems, srecv_sems,
    ):
        my_x = lax.axis_index("x")
        my_y = lax.axis_index("y")
        my_z = lax.axis_index("z")
        dst_y = pi_ref[my_y]
        src_y = jnp.int32(0)
        for k in range(4):
            src_y = jnp.where(pi_ref[k] == my_y, jnp.int32(k), src_y)

        barrier = pltpu.get_barrier_semaphore()
        pl.semaphore_signal(
            barrier, inc=1, device_id=(my_x, dst_y, my_z),
            device_id_type=pl.DeviceIdType.MESH,
        )
        pl.semaphore_signal(
            barrier, inc=1, device_id=(my_x, src_y, my_z),
            device_id_type=pl.DeviceIdType.MESH,
        )

        def chunk_rdma(c):
            sl = pl.ds(c * rows, rows)
            return pltpu.make_async_remote_copy(
                src_ref=send_q.at[sl],
                dst_ref=recv_q.at[sl],
                send_sem=qsend_sems.at[c],
                recv_sem=qrecv_sems.at[c],
                device_id=(my_x, dst_y, my_z),
                device_id_type=pl.DeviceIdType.MESH,
            )

        def scale_rdma(c):
            sl = pl.ds(c * 8, 8)
            return pltpu.make_async_remote_copy(
                src_ref=send_s.at[sl],
                dst_ref=recv_s.at[sl],
                send_sem=ssend_sems.at[c],
                recv_sem=srecv_sems.at[c],
                device_id=(my_x, dst_y, my_z),
                device_id_type=pl.DeviceIdType.MESH,
            )

        for c in range(N_CHUNKS):
            sl = pl.ds(c * rows, rows)
            xc = x_ref[0, sl, :]
            absmax = jnp.max(jnp.abs(xc))
            inv = jnp.where(absmax > 0, 127.0 / absmax, 0.0)
            send_q[sl, :] = jnp.round(xc * inv).astype(jnp.int8)
            send_s[pl.ds(c * 8, 8), :] = jnp.full(
                (8, 128), absmax * (1.0 / 127.0), jnp.float32
            )
            if c == 0:
                pl.semaphore_wait(barrier, 2)
            scale_rdma(c).start()
            chunk_rdma(c).start()

        for c in range(N_CHUNKS):
            sl = pl.ds(c * rows, rows)
            s_rdma = scale_rdma(c)
            s_rdma.wait_send()
            s_rdma.wait_recv()
            q_rdma = chunk_rdma(c)
            q_rdma.wait_send()
            q_rdma.wait_recv()
            sc = recv_s[pl.ds(c * 8, 1), pl.ds(0, 1)]
            out_ref[0, sl, :] = recv_q[sl, :].astype(jnp.float32) * sc

    return pl.pallas_call(
        body,
        out_shape=jax.ShapeDtypeStruct(x.shape, x.dtype),
        in_specs=[
            pl.BlockSpec(memory_space=pltpu.VMEM),
            pl.BlockSpec(memory_space=pltpu.SMEM),
        ],
        out_specs=pl.BlockSpec(memory_space=pltpu.VMEM),
        scratch_shapes=[
            pltpu.VMEM((m, n), jnp.int8),
            pltpu.VMEM((m, n), jnp.int8),
            pltpu.VMEM((N_CHUNKS * 8, 128), jnp.float32),
            pltpu.VMEM((N_CHUNKS * 8, 128), jnp.float32),
            pltpu.SemaphoreType.DMA((N_CHUNKS,)),
            pltpu.SemaphoreType.DMA((N_CHUNKS,)),
            pltpu.SemaphoreType.DMA((N_CHUNKS,)),
            pltpu.SemaphoreType.DMA((N_CHUNKS,)),
        ],
        compiler_params=pltpu.CompilerParams(collective_id=0),
    )(x, pi)
